# Initial kernel scaffold; baseline (speedup 1.0000x reference)
import jax
import jax.numpy as jnp
from jax.experimental import pallas as pl


def kernel(x, edge_index, W1, b1, W2, b2):
    raise NotImplementedError("write your pallas kernel here")


# speedup vs baseline: 32.0851x; 32.0851x over previous
import functools

import jax
import jax.numpy as jnp
from jax import lax
from jax.experimental import pallas as pl
from jax.experimental.pallas import tpu as pltpu
from jax.experimental.pallas import tpu_sc as plsc

N = 10000
DF = 128
NE = 320000
NC = 2
NS = 16
NW = NC * NS
CH = 128
NP = 10112
RPT = NP // NS
NE_PAD = 327680
NCHUNK = NE_PAD // CH
CPW = NCHUNK // NW

_mesh = plsc.VectorSubcoreMesh(
    core_axis_name="c", subcore_axis_name="s", num_cores=NC, num_subcores=NS)
_sc_params = pltpu.CompilerParams(use_tc_tiling_on_sc=False)


def _sc_scatter(d):
  @functools.partial(
      pl.kernel,
      out_type=jax.ShapeDtypeStruct((NC, NP, d), jnp.float32),
      mesh=_mesh,
      compiler_params=_sc_params,
      scratch_types=[
          pltpu.VMEM_SHARED((NP, d), jnp.float32),
          pltpu.VMEM((CPW, CH), jnp.int32),
          pltpu.VMEM((CPW, CH), jnp.int32),
          pltpu.VMEM((CH, d), jnp.float32),
          pltpu.VMEM((RPT, d), jnp.float32),
          pltpu.SemaphoreType.DMA,
      ],
  )
  def k(table, srcm, dstm, zrow, out, acc, sidx, didx, rows, stage, sem):
    c = lax.axis_index("c")
    s = lax.axis_index("s")
    wid = s * NC + c
    pltpu.sync_copy(zrow, stage)
    pltpu.sync_copy(stage, acc.at[pl.ds(s * RPT, RPT)])
    pltpu.sync_copy(srcm.at[pl.ds(wid * CPW, CPW)], sidx)
    pltpu.sync_copy(dstm.at[pl.ds(wid * CPW, CPW)], didx)
    plsc.subcore_barrier()

    def step(j, carry):
      pltpu.async_copy(table.at[sidx.at[j]], rows, sem).wait()
      pltpu.sync_copy(rows, acc.at[didx.at[j]], add=True)
      return carry

    lax.fori_loop(0, CPW, step, 0)
    plsc.subcore_barrier()
    pltpu.sync_copy(acc.at[pl.ds(s * RPT, RPT)], stage)
    pltpu.sync_copy(stage, out.at[c, pl.ds(s * RPT, RPT)])

  return k


@functools.partial(
    pl.kernel,
    out_type=jax.ShapeDtypeStruct((NC, NP, 8), jnp.float32),
    mesh=_mesh,
    compiler_params=_sc_params,
    scratch_types=[
        pltpu.VMEM_SHARED((NP, 8), jnp.float32),
        pltpu.VMEM((CPW, CH), jnp.int32),
        pltpu.VMEM((CH, 8), jnp.float32),
        pltpu.VMEM((RPT, 8), jnp.float32),
        pltpu.SemaphoreType.DMA,
    ],
)
def _sc_degree(dstm, onesrow, zrow, out, acc, didx, ones_v, stage, sem):
  c = lax.axis_index("c")
  s = lax.axis_index("s")
  wid = s * NC + c
  pltpu.sync_copy(zrow, stage)
  pltpu.sync_copy(stage, acc.at[pl.ds(s * RPT, RPT)])
  pltpu.sync_copy(onesrow, ones_v)
  pltpu.sync_copy(dstm.at[pl.ds(wid * CPW, CPW)], didx)
  plsc.subcore_barrier()

  def step(j, carry):
    pltpu.sync_copy(ones_v, acc.at[didx.at[j]], add=True)
    return carry

  lax.fori_loop(0, CPW, step, 0)
  plsc.subcore_barrier()
  pltpu.sync_copy(acc.at[pl.ds(s * RPT, RPT)], stage)
  pltpu.sync_copy(stage, out.at[c, pl.ds(s * RPT, RPT)])


def _tc_pre(degp_ref, x_ref, w1_ref, u_ref, g1_ref):
  deg = degp_ref[0, :, 0:1] + degp_ref[1, :, 0:1] + 1.0
  u = lax.rsqrt(deg)
  u_ref[...] = u
  g1_ref[...] = jnp.dot(x_ref[...], w1_ref[...],
                        preferred_element_type=jnp.float32) * u


def _tc_mid(a1_ref, g1_ref, u_ref, b1_ref, w2_ref, g2_ref):
  u = u_ref[...]
  h = u * (a1_ref[0] + a1_ref[1] + g1_ref[...]) + b1_ref[...]
  hp = jnp.maximum(h, 0.0)
  g2_ref[...] = jnp.dot(hp, w2_ref[...],
                        preferred_element_type=jnp.float32) * u


def _tc_post(a2_ref, g2_ref, u_ref, b2_ref, o_ref):
  u = u_ref[...]
  o = u * (a2_ref[0, :, 0:2] + a2_ref[1, :, 0:2] + g2_ref[:, 0:2]) + b2_ref[...]
  m = jnp.max(o, axis=1, keepdims=True)
  e = jnp.exp(o - m)
  o_ref[...] = (o - m) - jnp.log(jnp.sum(e, axis=1, keepdims=True))


_scatter8 = _sc_scatter(8)


def kernel(x, edge_index, W1, b1, W2, b2):
  ei = edge_index.astype(jnp.int32)
  pad = jnp.full((NE_PAD - NE,), N, dtype=jnp.int32)
  srcm = jnp.concatenate([ei[0], pad]).reshape(NCHUNK, CH)
  dstm = jnp.concatenate([ei[1], pad]).reshape(NCHUNK, CH)
  x_pad = jnp.pad(x, ((0, NP - N), (0, 0)))
  onesrow = jnp.ones((CH, 8), jnp.float32)
  z8 = jnp.zeros((RPT, 8), jnp.float32)

  degp = _sc_degree(dstm, onesrow, z8)

  u, g1 = pl.pallas_call(
      _tc_pre,
      out_shape=[jax.ShapeDtypeStruct((NP, 1), jnp.float32),
                 jax.ShapeDtypeStruct((NP, 8), jnp.float32)],
  )(degp, x_pad, W1)

  a1 = _scatter8(g1, srcm, dstm, z8)

  W2p = jnp.pad(W2, ((0, 0), (0, 6)))
  g2 = pl.pallas_call(
      _tc_mid,
      out_shape=jax.ShapeDtypeStruct((NP, 8), jnp.float32),
  )(a1, g1, u, b1.reshape(1, 8), W2p)

  a2 = _scatter8(g2, srcm, dstm, z8)

  o = pl.pallas_call(
      _tc_post,
      out_shape=jax.ShapeDtypeStruct((NP, 2), jnp.float32),
  )(a2, g2, u, b2.reshape(1, 2))

  return o[:N]


# speedup vs baseline: 53.7970x; 1.6767x over previous
import functools

import jax
import jax.numpy as jnp
from jax import lax
from jax.experimental import pallas as pl
from jax.experimental.pallas import tpu as pltpu
from jax.experimental.pallas import tpu_sc as plsc

N = 10000
DF = 128
NE = 320000
NC = 2
NS = 16
NW = NC * NS
CH = 128
NP = 10112
RPT = NP // NS
NE_PAD = 327680
NCHUNK = NE_PAD // CH
CPW = NCHUNK // NW
KB = 4

_mesh = plsc.VectorSubcoreMesh(
    core_axis_name="c", subcore_axis_name="s", num_cores=NC, num_subcores=NS)
_sc_params = pltpu.CompilerParams(use_tc_tiling_on_sc=False)


def _sc_scatter(d):
  @functools.partial(
      pl.kernel,
      out_type=jax.ShapeDtypeStruct((NC, NP, d), jnp.float32),
      mesh=_mesh,
      compiler_params=_sc_params,
      scratch_types=[
          pltpu.VMEM_SHARED((NP, d), jnp.float32),
          pltpu.VMEM_SHARED((NP, d), jnp.float32),
          pltpu.VMEM((CPW, CH), jnp.int32),
          pltpu.VMEM((CPW, CH), jnp.int32),
          pltpu.VMEM((KB, CH, d), jnp.float32),
          pltpu.VMEM((RPT, d), jnp.float32),
          pltpu.SemaphoreType.DMA,
          pltpu.SemaphoreType.DMA,
      ],
  )
  def k(table, srcm, dstm, zrow, out, acc, tab_s, sidx, didx, rows, stage,
        gsem, ssem):
    c = lax.axis_index("c")
    s = lax.axis_index("s")
    wid = s * NC + c
    pltpu.sync_copy(zrow, stage)
    pltpu.sync_copy(stage, acc.at[pl.ds(s * RPT, RPT)])
    pltpu.sync_copy(table.at[pl.ds(s * RPT, RPT)], stage)
    pltpu.sync_copy(stage, tab_s.at[pl.ds(s * RPT, RPT)])
    pltpu.sync_copy(srcm.at[pl.ds(wid * CPW, CPW)], sidx)
    pltpu.sync_copy(dstm.at[pl.ds(wid * CPW, CPW)], didx)
    plsc.subcore_barrier()

    def step(w, carry):
      gd = [pltpu.async_copy(tab_s.at[sidx.at[w * KB + b]], rows.at[b], gsem)
            for b in range(KB)]
      for t in gd:
        t.wait()
      sd = [pltpu.async_copy(rows.at[b], acc.at[didx.at[w * KB + b]], ssem,
                             add=True)
            for b in range(KB)]
      for t in sd:
        t.wait()
      return carry

    lax.fori_loop(0, CPW // KB, step, 0)
    plsc.subcore_barrier()
    pltpu.sync_copy(acc.at[pl.ds(s * RPT, RPT)], stage)
    pltpu.sync_copy(stage, out.at[c, pl.ds(s * RPT, RPT)])

  return k


@functools.partial(
    pl.kernel,
    out_type=jax.ShapeDtypeStruct((NC, NP, 8), jnp.float32),
    mesh=_mesh,
    compiler_params=_sc_params,
    scratch_types=[
        pltpu.VMEM_SHARED((NP, 8), jnp.float32),
        pltpu.VMEM((CPW, CH), jnp.int32),
        pltpu.VMEM((CH, 8), jnp.float32),
        pltpu.VMEM((RPT, 8), jnp.float32),
        pltpu.SemaphoreType.DMA,
    ],
)
def _sc_degree(dstm, onesrow, zrow, out, acc, didx, ones_v, stage, sem):
  c = lax.axis_index("c")
  s = lax.axis_index("s")
  wid = s * NC + c
  pltpu.sync_copy(zrow, stage)
  pltpu.sync_copy(stage, acc.at[pl.ds(s * RPT, RPT)])
  pltpu.sync_copy(onesrow, ones_v)
  pltpu.sync_copy(dstm.at[pl.ds(wid * CPW, CPW)], didx)
  plsc.subcore_barrier()

  def step(w, carry):
    sd = [pltpu.async_copy(ones_v, acc.at[didx.at[w * 8 + b]], sem, add=True)
          for b in range(8)]
    for t in sd:
      t.wait()
    return carry

  lax.fori_loop(0, CPW // 8, step, 0)
  plsc.subcore_barrier()
  pltpu.sync_copy(acc.at[pl.ds(s * RPT, RPT)], stage)
  pltpu.sync_copy(stage, out.at[c, pl.ds(s * RPT, RPT)])


def _tc_pre(degp_ref, x_ref, w1_ref, u_ref, g1_ref):
  deg = degp_ref[0, :, 0:1] + degp_ref[1, :, 0:1] + 1.0
  u = lax.rsqrt(deg)
  u_ref[...] = u
  g1_ref[...] = jnp.dot(x_ref[...], w1_ref[...],
                        preferred_element_type=jnp.float32) * u


def _tc_mid(a1_ref, g1_ref, u_ref, b1_ref, w2_ref, g2_ref):
  u = u_ref[...]
  h = u * (a1_ref[0] + a1_ref[1] + g1_ref[...]) + b1_ref[...]
  hp = jnp.maximum(h, 0.0)
  g2_ref[...] = jnp.dot(hp, w2_ref[...],
                        preferred_element_type=jnp.float32) * u


def _tc_post(a2_ref, g2_ref, u_ref, b2_ref, o_ref):
  u = u_ref[...]
  o = u * (a2_ref[0, :, 0:2] + a2_ref[1, :, 0:2] + g2_ref[:, 0:2]) + b2_ref[...]
  m = jnp.max(o, axis=1, keepdims=True)
  e = jnp.exp(o - m)
  o_ref[...] = (o - m) - jnp.log(jnp.sum(e, axis=1, keepdims=True))


_scatter8 = _sc_scatter(8)


def kernel(x, edge_index, W1, b1, W2, b2):
  ei = edge_index.astype(jnp.int32)
  pad = jnp.full((NE_PAD - NE,), N, dtype=jnp.int32)
  srcm = jnp.concatenate([ei[0], pad]).reshape(NCHUNK, CH)
  dstm = jnp.concatenate([ei[1], pad]).reshape(NCHUNK, CH)
  x_pad = jnp.pad(x, ((0, NP - N), (0, 0)))
  onesrow = jnp.ones((CH, 8), jnp.float32)
  z8 = jnp.zeros((RPT, 8), jnp.float32)

  degp = _sc_degree(dstm, onesrow, z8)

  u, g1 = pl.pallas_call(
      _tc_pre,
      out_shape=[jax.ShapeDtypeStruct((NP, 1), jnp.float32),
                 jax.ShapeDtypeStruct((NP, 8), jnp.float32)],
  )(degp, x_pad, W1)

  a1 = _scatter8(g1, srcm, dstm, z8)

  W2p = jnp.pad(W2, ((0, 0), (0, 6)))
  g2 = pl.pallas_call(
      _tc_mid,
      out_shape=jax.ShapeDtypeStruct((NP, 8), jnp.float32),
  )(a1, g1, u, b1.reshape(1, 8), W2p)

  a2 = _scatter8(g2, srcm, dstm, z8)

  o = pl.pallas_call(
      _tc_post,
      out_shape=jax.ShapeDtypeStruct((NP, 2), jnp.float32),
  )(a2, g2, u, b2.reshape(1, 2))

  return o[:N]


# speedup vs baseline: 56.4814x; 1.0499x over previous
import functools

import jax
import jax.numpy as jnp
from jax import lax
from jax.experimental import pallas as pl
from jax.experimental.pallas import tpu as pltpu
from jax.experimental.pallas import tpu_sc as plsc

N = 10000
DF = 128
NE = 320000
NC = 2
NS = 16
NW = NC * NS
CH = 128
NP = 10112
RPT = NP // NS
NE_PAD = 327680
NCHUNK = NE_PAD // CH
CPW = NCHUNK // NW
KB = 8

_mesh = plsc.VectorSubcoreMesh(
    core_axis_name="c", subcore_axis_name="s", num_cores=NC, num_subcores=NS)
_sc_params = pltpu.CompilerParams(use_tc_tiling_on_sc=False)


def _sc_scatter(d):
  @functools.partial(
      pl.kernel,
      out_type=jax.ShapeDtypeStruct((NC, NP, d), jnp.float32),
      mesh=_mesh,
      compiler_params=_sc_params,
      scratch_types=[
          pltpu.VMEM_SHARED((NP, d), jnp.float32),
          pltpu.VMEM_SHARED((NP, d), jnp.float32),
          pltpu.VMEM((CPW, CH), jnp.int32),
          pltpu.VMEM((CPW, CH), jnp.int32),
          pltpu.VMEM((KB, CH, d), jnp.float32),
          pltpu.VMEM((RPT, d), jnp.float32),
          pltpu.SemaphoreType.DMA,
          pltpu.SemaphoreType.DMA,
      ],
  )
  def k(table, srcm, dstm, zrow, out, acc, tab_s, sidx, didx, rows, stage,
        gsem, ssem):
    c = lax.axis_index("c")
    s = lax.axis_index("s")
    wid = s * NC + c
    pltpu.sync_copy(zrow, stage)
    pltpu.sync_copy(stage, acc.at[pl.ds(s * RPT, RPT)])
    pltpu.sync_copy(table.at[pl.ds(s * RPT, RPT)], stage)
    pltpu.sync_copy(stage, tab_s.at[pl.ds(s * RPT, RPT)])
    pltpu.sync_copy(srcm.at[pl.ds(wid * CPW, CPW)], sidx)
    pltpu.sync_copy(dstm.at[pl.ds(wid * CPW, CPW)], didx)
    plsc.subcore_barrier()

    def step(w, carry):
      gd = [pltpu.async_copy(tab_s.at[sidx.at[w * KB + b]], rows.at[b], gsem)
            for b in range(KB)]
      sd = []
      for b in range(KB):
        gd[b].wait()
        sd.append(pltpu.async_copy(rows.at[b], acc.at[didx.at[w * KB + b]],
                                   ssem, add=True))
      for t in sd:
        t.wait()
      return carry

    lax.fori_loop(0, CPW // KB, step, 0)
    plsc.subcore_barrier()
    pltpu.sync_copy(acc.at[pl.ds(s * RPT, RPT)], stage)
    pltpu.sync_copy(stage, out.at[c, pl.ds(s * RPT, RPT)])

  return k


@functools.partial(
    pl.kernel,
    out_type=jax.ShapeDtypeStruct((NC, NP, 8), jnp.float32),
    mesh=_mesh,
    compiler_params=_sc_params,
    scratch_types=[
        pltpu.VMEM_SHARED((NP, 8), jnp.float32),
        pltpu.VMEM((CPW, CH), jnp.int32),
        pltpu.VMEM((CH, 8), jnp.float32),
        pltpu.VMEM((RPT, 8), jnp.float32),
        pltpu.SemaphoreType.DMA,
    ],
)
def _sc_degree(dstm, onesrow, zrow, out, acc, didx, ones_v, stage, sem):
  c = lax.axis_index("c")
  s = lax.axis_index("s")
  wid = s * NC + c
  pltpu.sync_copy(zrow, stage)
  pltpu.sync_copy(stage, acc.at[pl.ds(s * RPT, RPT)])
  pltpu.sync_copy(onesrow, ones_v)
  pltpu.sync_copy(dstm.at[pl.ds(wid * CPW, CPW)], didx)
  plsc.subcore_barrier()

  def step(w, carry):
    sd = [pltpu.async_copy(ones_v, acc.at[didx.at[w * 8 + b]], sem, add=True)
          for b in range(8)]
    for t in sd:
      t.wait()
    return carry

  lax.fori_loop(0, CPW // 8, step, 0)
  plsc.subcore_barrier()
  pltpu.sync_copy(acc.at[pl.ds(s * RPT, RPT)], stage)
  pltpu.sync_copy(stage, out.at[c, pl.ds(s * RPT, RPT)])


def _tc_pre(degp_ref, x_ref, w1_ref, u_ref, g1_ref):
  deg = degp_ref[0, :, 0:1] + degp_ref[1, :, 0:1] + 1.0
  u = lax.rsqrt(deg)
  u_ref[...] = u
  g1_ref[...] = jnp.dot(x_ref[...], w1_ref[...],
                        preferred_element_type=jnp.float32) * u


def _tc_mid(a1_ref, g1_ref, u_ref, b1_ref, w2_ref, g2_ref):
  u = u_ref[...]
  h = u * (a1_ref[0] + a1_ref[1] + g1_ref[...]) + b1_ref[...]
  hp = jnp.maximum(h, 0.0)
  g2_ref[...] = jnp.dot(hp, w2_ref[...],
                        preferred_element_type=jnp.float32) * u


def _tc_post(a2_ref, g2_ref, u_ref, b2_ref, o_ref):
  u = u_ref[...]
  o = u * (a2_ref[0, :, 0:2] + a2_ref[1, :, 0:2] + g2_ref[:, 0:2]) + b2_ref[...]
  m = jnp.max(o, axis=1, keepdims=True)
  e = jnp.exp(o - m)
  o_ref[...] = (o - m) - jnp.log(jnp.sum(e, axis=1, keepdims=True))


_scatter8 = _sc_scatter(8)


def kernel(x, edge_index, W1, b1, W2, b2):
  ei = edge_index.astype(jnp.int32)
  pad = jnp.full((NE_PAD - NE,), N, dtype=jnp.int32)
  srcm = jnp.concatenate([ei[0], pad]).reshape(NCHUNK, CH)
  dstm = jnp.concatenate([ei[1], pad]).reshape(NCHUNK, CH)
  x_pad = jnp.pad(x, ((0, NP - N), (0, 0)))
  onesrow = jnp.ones((CH, 8), jnp.float32)
  z8 = jnp.zeros((RPT, 8), jnp.float32)

  degp = _sc_degree(dstm, onesrow, z8)

  u, g1 = pl.pallas_call(
      _tc_pre,
      out_shape=[jax.ShapeDtypeStruct((NP, 1), jnp.float32),
                 jax.ShapeDtypeStruct((NP, 8), jnp.float32)],
  )(degp, x_pad, W1)

  a1 = _scatter8(g1, srcm, dstm, z8)

  W2p = jnp.pad(W2, ((0, 0), (0, 6)))
  g2 = pl.pallas_call(
      _tc_mid,
      out_shape=jax.ShapeDtypeStruct((NP, 8), jnp.float32),
  )(a1, g1, u, b1.reshape(1, 8), W2p)

  a2 = _scatter8(g2, srcm, dstm, z8)

  o = pl.pallas_call(
      _tc_post,
      out_shape=jax.ShapeDtypeStruct((NP, 2), jnp.float32),
  )(a2, g2, u, b2.reshape(1, 2))

  return o[:N]


# speedup vs baseline: 69.4699x; 1.2300x over previous
import functools

import jax
import jax.numpy as jnp
from jax import lax
from jax.experimental import pallas as pl
from jax.experimental.pallas import tpu as pltpu
from jax.experimental.pallas import tpu_sc as plsc

N = 10000
DF = 128
NE = 320000
NC = 2
NS = 16
NW = NC * NS
CH = 128
NP = 10112
RPT = NP // NS
NCHUNK = NE // CH
CPW0 = NCHUNK // NW
NEXTRA = NCHUNK - CPW0 * NW
CPmax = CPW0 + 1
KB = 8

_mesh = plsc.VectorSubcoreMesh(
    core_axis_name="c", subcore_axis_name="s", num_cores=NC, num_subcores=NS)
_sc_params = pltpu.CompilerParams(use_tc_tiling_on_sc=False)


def _chunk_window(wid):
  start = CPW0 * wid + jnp.minimum(wid, NEXTRA)
  n = CPW0 + jnp.where(wid < NEXTRA, 1, 0)
  base = jnp.minimum(start, NCHUNK - CPmax)
  return start - base, n, base


def _sc_scatter(d):
  @functools.partial(
      pl.kernel,
      out_type=jax.ShapeDtypeStruct((NC, NP, d), jnp.float32),
      mesh=_mesh,
      compiler_params=_sc_params,
      scratch_types=[
          pltpu.VMEM_SHARED((NP, d), jnp.float32),
          pltpu.VMEM_SHARED((NP, d), jnp.float32),
          pltpu.VMEM((2, CPmax, CH), jnp.int32),
          pltpu.VMEM((KB, CH, d), jnp.float32),
          pltpu.VMEM((RPT, d), jnp.float32),
          pltpu.SemaphoreType.DMA,
          pltpu.SemaphoreType.DMA,
      ],
  )
  def k(table, eim, zrow, out, acc, tab_s, sdidx, rows, stage, gsem, ssem):
    c = lax.axis_index("c")
    s = lax.axis_index("s")
    wid = s * NC + c
    off, n, base = _chunk_window(wid)
    pltpu.sync_copy(zrow, stage)
    pltpu.sync_copy(stage, acc.at[pl.ds(s * RPT, RPT)])
    pltpu.sync_copy(table.at[pl.ds(s * RPT, RPT)], stage)
    pltpu.sync_copy(stage, tab_s.at[pl.ds(s * RPT, RPT)])
    pltpu.sync_copy(eim.at[:, pl.ds(base, CPmax)], sdidx)
    plsc.subcore_barrier()

    def step(w, carry):
      j = off + w * KB
      gd = [pltpu.async_copy(tab_s.at[sdidx.at[0, j + b]], rows.at[b], gsem)
            for b in range(KB)]
      sd = []
      for b in range(KB):
        gd[b].wait()
        sd.append(pltpu.async_copy(rows.at[b], acc.at[sdidx.at[1, j + b]],
                                   ssem, add=True))
      for t in sd:
        t.wait()
      return carry

    nb = n // KB
    lax.fori_loop(0, nb, step, 0, unroll=False)

    def tail(w, carry):
      j = off + w
      pltpu.async_copy(tab_s.at[sdidx.at[0, j]], rows.at[0], gsem).wait()
      pltpu.async_copy(rows.at[0], acc.at[sdidx.at[1, j]], ssem,
                       add=True).wait()
      return carry

    lax.fori_loop(nb * KB, n, tail, 0, unroll=False)
    plsc.subcore_barrier()
    pltpu.sync_copy(acc.at[pl.ds(s * RPT, RPT)], stage)
    pltpu.sync_copy(stage, out.at[c, pl.ds(s * RPT, RPT)])

  return k


@functools.partial(
    pl.kernel,
    out_type=jax.ShapeDtypeStruct((NC, NP, 8), jnp.float32),
    mesh=_mesh,
    compiler_params=_sc_params,
    scratch_types=[
        pltpu.VMEM_SHARED((NP, 8), jnp.float32),
        pltpu.VMEM((CPmax, CH), jnp.int32),
        pltpu.VMEM((CH, 8), jnp.float32),
        pltpu.VMEM((RPT, 8), jnp.float32),
        pltpu.SemaphoreType.DMA,
    ],
)
def _sc_degree(eim, onesrow, zrow, out, acc, didx, ones_v, stage, sem):
  c = lax.axis_index("c")
  s = lax.axis_index("s")
  wid = s * NC + c
  off, n, base = _chunk_window(wid)
  pltpu.sync_copy(zrow, stage)
  pltpu.sync_copy(stage, acc.at[pl.ds(s * RPT, RPT)])
  pltpu.sync_copy(onesrow, ones_v)
  pltpu.sync_copy(eim.at[1, pl.ds(base, CPmax)], didx)
  plsc.subcore_barrier()

  def step(w, carry):
    j = off + w * KB
    sd = [pltpu.async_copy(ones_v, acc.at[didx.at[j + b]], sem, add=True)
          for b in range(KB)]
    for t in sd:
      t.wait()
    return carry

  nb = n // KB
  lax.fori_loop(0, nb, step, 0, unroll=False)

  def tail(w, carry):
    pltpu.async_copy(ones_v, acc.at[didx.at[off + w]], sem, add=True).wait()
    return carry

  lax.fori_loop(nb * KB, n, tail, 0, unroll=False)
  plsc.subcore_barrier()
  pltpu.sync_copy(acc.at[pl.ds(s * RPT, RPT)], stage)
  pltpu.sync_copy(stage, out.at[c, pl.ds(s * RPT, RPT)])


def _tc_pre(degp_ref, x_ref, w1_ref, u_ref, g1_ref):
  deg = degp_ref[0, :, 0:1] + degp_ref[1, :, 0:1] + 1.0
  u = lax.rsqrt(deg)
  u_ref[...] = u
  g1_ref[0:N, :] = jnp.dot(x_ref[...], w1_ref[...],
                           preferred_element_type=jnp.float32) * u[0:N]
  g1_ref[N:NP, :] = jnp.zeros((NP - N, 8), jnp.float32)


def _tc_mid(a1_ref, g1_ref, u_ref, b1_ref, w2_ref, g2_ref):
  u = u_ref[...]
  h = u * (a1_ref[0] + a1_ref[1] + g1_ref[...]) + b1_ref[...]
  hp = jnp.maximum(h, 0.0)
  w2p = jnp.pad(w2_ref[...], ((0, 0), (0, 6)))
  g2_ref[...] = jnp.dot(hp, w2p, preferred_element_type=jnp.float32) * u


def _tc_post(a2_ref, g2_ref, u_ref, b2_ref, o_ref):
  u = u_ref[0:N]
  o = (u * (a2_ref[0, 0:N, 0:2] + a2_ref[1, 0:N, 0:2] + g2_ref[0:N, 0:2])
       + b2_ref[...])
  m = jnp.max(o, axis=1, keepdims=True)
  e = jnp.exp(o - m)
  o_ref[...] = (o - m) - jnp.log(jnp.sum(e, axis=1, keepdims=True))


_scatter8 = _sc_scatter(8)


def kernel(x, edge_index, W1, b1, W2, b2):
  eim = edge_index.astype(jnp.int32).reshape(2, NCHUNK, CH)
  onesrow = jnp.ones((CH, 8), jnp.float32)
  z8 = jnp.zeros((RPT, 8), jnp.float32)

  degp = _sc_degree(eim, onesrow, z8)

  u, g1 = pl.pallas_call(
      _tc_pre,
      out_shape=[jax.ShapeDtypeStruct((NP, 1), jnp.float32),
                 jax.ShapeDtypeStruct((NP, 8), jnp.float32)],
  )(degp, x, W1)

  a1 = _scatter8(g1, eim, z8)

  g2 = pl.pallas_call(
      _tc_mid,
      out_shape=jax.ShapeDtypeStruct((NP, 8), jnp.float32),
  )(a1, g1, u, b1.reshape(1, 8), W2)

  a2 = _scatter8(g2, eim, z8)

  o = pl.pallas_call(
      _tc_post,
      out_shape=jax.ShapeDtypeStruct((N, 2), jnp.float32),
  )(a2, g2, u, b2.reshape(1, 2))

  return o
